# baseline (speedup 1.0000x reference)
import jax
import jax.numpy as jnp
from jax.experimental import pallas as pl


def _sage_body(adj1_ref, adj2_ref, xe1_ref, xe2_ref, w_ref, b_ref, out_ref):
    in_f = w_ref.shape[1]
    m1 = adj1_ref[...][:, 0, 0, :].astype(jnp.bfloat16)
    m2 = adj2_ref[...][:, 0, 0, :].astype(jnp.bfloat16)
    accw = (jnp.dot(m1, xe1_ref[...], preferred_element_type=jnp.float32)
            + jnp.dot(m2, xe2_ref[...], preferred_element_type=jnp.float32))
    acc = accw[:, :in_f]
    deg = accw[:, in_f:in_f + 1]
    agg = acc / deg
    out_ref[...] = jax.lax.dot_general(
        agg, w_ref[...], (((1,), (1,)), ((), ())),
        preferred_element_type=jnp.float32) + b_ref[...]


def kernel(input, adjacency_matrix, W, b):
    n, in_f = input.shape
    out_f = W.shape[0]
    bm = 400
    xe = jnp.concatenate(
        [input,
         jnp.ones((n, 1), input.dtype),
         jnp.zeros((n, 127), input.dtype)], axis=1).astype(jnp.bfloat16)
    b2 = b.reshape(1, out_f)
    adj4 = adjacency_matrix.reshape(n, 2, 1, n // 2)
    return pl.pallas_call(
        _sage_body,
        out_shape=jax.ShapeDtypeStruct((n, out_f), jnp.float32),
        grid=(n // bm,),
        in_specs=[
            pl.BlockSpec((bm, 1, 1, n // 2), lambda i: (i, 0, 0, 0)),
            pl.BlockSpec((bm, 1, 1, n // 2), lambda i: (i, 1, 0, 0)),
            pl.BlockSpec((n // 2, in_f + 128), lambda i: (0, 0)),
            pl.BlockSpec((n // 2, in_f + 128), lambda i: (1, 0)),
            pl.BlockSpec((out_f, in_f), lambda i: (0, 0)),
            pl.BlockSpec((1, out_f), lambda i: (0, 0)),
        ],
        out_specs=pl.BlockSpec((bm, out_f), lambda i: (i, 0)),
    )(adj4, adj4, xe, xe, W, b2)


# speedup vs baseline: 22.5940x; 22.5940x over previous
import jax
import jax.numpy as jnp
from jax.experimental import pallas as pl


def _sage_body(adj1_ref, adj2_ref, xe_ref, w_ref, b_ref, o1_ref, o2_ref):
    in_f = w_ref.shape[1]
    xe = xe_ref[...]
    w = w_ref[...]
    bvec = b_ref[...]
    for adj_ref, o_ref in ((adj1_ref, o1_ref), (adj2_ref, o2_ref)):
        mask = adj_ref[...].astype(jnp.bfloat16)
        accw = jnp.dot(mask, xe, preferred_element_type=jnp.float32)
        agg = accw[:, :in_f] / accw[:, in_f:in_f + 1]
        o_ref[...] = jax.lax.dot_general(
            agg, w, (((1,), (1,)), ((), ())),
            preferred_element_type=jnp.float32) + bvec


def kernel(input, adjacency_matrix, W, b):
    n, in_f = input.shape
    out_f = W.shape[0]
    bm = 200
    nsteps = n // (2 * bm)
    xe = jnp.concatenate(
        [input,
         jnp.ones((n, 1), input.dtype),
         jnp.zeros((n, 127), input.dtype)], axis=1).astype(jnp.bfloat16)
    b2 = b.reshape(1, out_f)
    o1, o2 = pl.pallas_call(
        _sage_body,
        out_shape=[jax.ShapeDtypeStruct((n // 2, out_f), jnp.float32),
                   jax.ShapeDtypeStruct((n // 2, out_f), jnp.float32)],
        grid=(nsteps,),
        in_specs=[
            pl.BlockSpec((bm, n), lambda i: (i, 0)),
            pl.BlockSpec((bm, n), lambda i, _m=nsteps: (i + _m, 0)),
            pl.BlockSpec((n, in_f + 128), lambda i: (0, 0)),
            pl.BlockSpec((out_f, in_f), lambda i: (0, 0)),
            pl.BlockSpec((1, out_f), lambda i: (0, 0)),
        ],
        out_specs=[pl.BlockSpec((bm, out_f), lambda i: (i, 0)),
                   pl.BlockSpec((bm, out_f), lambda i: (i, 0))],
    )(adjacency_matrix, adjacency_matrix, xe, W, b2)
    return jnp.concatenate([o1, o2], axis=0)


# speedup vs baseline: 25.8604x; 1.1446x over previous
import jax
import jax.numpy as jnp
from jax.experimental import pallas as pl
from jax.experimental.pallas import tpu as pltpu


def _sage_body(adj_ref, xe_ref, w_ref, b_ref, out_ref):
    in_f = w_ref.shape[1]
    mask = adj_ref[...].astype(jnp.bfloat16)
    accw = jnp.dot(mask, xe_ref[...], preferred_element_type=jnp.float32)
    agg = accw[:, :in_f] / accw[:, in_f:in_f + 1]
    out_ref[...] = jax.lax.dot_general(
        agg, w_ref[...], (((1,), (1,)), ((), ())),
        preferred_element_type=jnp.float32) + b_ref[...]


def kernel(input, adjacency_matrix, W, b):
    n, in_f = input.shape
    out_f = W.shape[0]
    bm = 480
    xe = jnp.concatenate(
        [input,
         jnp.ones((n, 1), input.dtype),
         jnp.zeros((n, 127), input.dtype)], axis=1).astype(jnp.bfloat16)
    b2 = b.reshape(1, out_f)
    return pl.pallas_call(
        _sage_body,
        out_shape=jax.ShapeDtypeStruct((n, out_f), jnp.float32),
        grid=(pl.cdiv(n, bm),),
        in_specs=[
            pl.BlockSpec((bm, n), lambda i: (i, 0)),
            pl.BlockSpec((n, in_f + 128), lambda i: (0, 0)),
            pl.BlockSpec((out_f, in_f), lambda i: (0, 0)),
            pl.BlockSpec((1, out_f), lambda i: (0, 0)),
        ],
        out_specs=pl.BlockSpec((bm, out_f), lambda i: (i, 0)),
        compiler_params=pltpu.CompilerParams(vmem_limit_bytes=63 * 2**20),
    )(adjacency_matrix, xe, W, b2)


# speedup vs baseline: 26.2781x; 1.0162x over previous
import jax
import jax.numpy as jnp
from jax.experimental import pallas as pl


def _sage_body(adj_ref, xe_ref, w_ref, b_ref, out_ref):
    in_f = w_ref.shape[1]
    mask = adj_ref[...].astype(jnp.bfloat16)
    accw = jnp.dot(mask, xe_ref[...], preferred_element_type=jnp.float32)
    agg = accw[:, :in_f] / accw[:, in_f:in_f + 1]
    out_ref[...] = jax.lax.dot_general(
        agg, w_ref[...], (((1,), (1,)), ((), ())),
        preferred_element_type=jnp.float32) + b_ref[...]


def kernel(input, adjacency_matrix, W, b):
    n, in_f = input.shape
    out_f = W.shape[0]
    bm = 400
    xe = jnp.concatenate(
        [input,
         jnp.ones((n, 1), input.dtype),
         jnp.zeros((n, 127), input.dtype)], axis=1).astype(jnp.bfloat16)
    b2 = b.reshape(1, out_f)
    return pl.pallas_call(
        _sage_body,
        out_shape=jax.ShapeDtypeStruct((n, out_f), jnp.float32),
        grid=(n // bm,),
        in_specs=[
            pl.BlockSpec((bm, n), lambda i: (i, 0)),
            pl.BlockSpec((n, in_f + 128), lambda i: (0, 0)),
            pl.BlockSpec((out_f, in_f), lambda i: (0, 0)),
            pl.BlockSpec((1, out_f), lambda i: (0, 0)),
        ],
        out_specs=pl.BlockSpec((bm, out_f), lambda i: (i, 0)),
    )(adjacency_matrix, xe, W, b2)
